# baseline (speedup 1.0000x reference)
import functools

import jax
import jax.numpy as jnp
from jax import lax
from jax.experimental import pallas as pl
from jax.experimental.pallas import tpu as pltpu
from jax.experimental.pallas import tpu_sc as plsc

_B = 4096
_SEQ = 50
_D = 128
_NC = 2
_NS = 16
_NW = _NC * _NS
_NSPLIT = 1
_BH = _B // _NSPLIT
_BPW = _BH // _NW
_CHUNK = 8
_STEPS = _BPW // _CHUNK
_G = _CHUNK * _SEQ
_NLANE = _D // 16


def _gather_mean(idx_flat, table):
    mesh = plsc.VectorSubcoreMesh(core_axis_name="c", subcore_axis_name="s")

    @functools.partial(
        pl.kernel,
        mesh=mesh,
        out_type=jax.ShapeDtypeStruct((_BH, _D), jnp.float32),
        scratch_types=[
            pltpu.VMEM((_G,), jnp.int32),
            pltpu.VMEM((_G,), jnp.int32),
            pltpu.VMEM((_G, _D), jnp.float32),
            pltpu.VMEM((_G, _D), jnp.float32),
            pltpu.VMEM((_CHUNK, _D), jnp.float32),
            pltpu.SemaphoreType.DMA,
            pltpu.SemaphoreType.DMA,
        ],
    )
    def k(idx_hbm, table_hbm, out_hbm, idx0, idx1, rows0, rows1, pooled_v,
          sem0, sem1):
        wid = lax.axis_index("s") * _NC + lax.axis_index("c")
        base = wid * _BPW

        def issue(s, idx_v, rows_v, sem):
            row0 = base + s * _CHUNK
            pltpu.sync_copy(idx_hbm.at[pl.ds(row0 * _SEQ, _G)], idx_v)
            pltpu.async_copy(table_hbm.at[idx_v], rows_v, sem)

        def wait(idx_v, rows_v, sem):
            pltpu.make_async_copy(table_hbm.at[idx_v], rows_v, sem).wait()

        def reduce_store(s, rows_v):
            for r in range(_CHUNK):
                def red(j2, acc):
                    j = 2 * j2
                    return tuple(
                        acc[c]
                        + rows_v[r * _SEQ + j, pl.ds(c * 16, 16)]
                        + rows_v[r * _SEQ + j + 1, pl.ds(c * 16, 16)]
                        for c in range(_NLANE)
                    )
                acc0 = tuple(jnp.zeros((16,), jnp.float32) for _ in range(_NLANE))
                acc = lax.fori_loop(0, _SEQ // 2, red, acc0)
                for c in range(_NLANE):
                    pooled_v[r, pl.ds(c * 16, 16)] = acc[c] * (1.0 / _SEQ)
            pltpu.sync_copy(pooled_v, out_hbm.at[pl.ds(base + s * _CHUNK, _CHUNK)])

        issue(0, idx0, rows0, sem0)

        def pair(i, carry):
            s = 2 * i
            issue(s + 1, idx1, rows1, sem1)
            wait(idx0, rows0, sem0)
            reduce_store(s, rows0)

            @pl.when(s + 2 < _STEPS)
            def _():
                issue(s + 2, idx0, rows0, sem0)

            wait(idx1, rows1, sem1)
            reduce_store(s + 1, rows1)
            return carry

        lax.fori_loop(0, _STEPS // 2, pair, 0)

    return k(idx_flat, table)


def _linear(pooled, W, b):
    def mm(p_ref, w_ref, b_ref, o_ref):
        o_ref[...] = lax.dot_general(
            p_ref[...], w_ref[...], (((1,), (1,)), ((), ())),
            preferred_element_type=jnp.float32,
        ) + b_ref[...]

    return pl.pallas_call(
        mm,
        out_shape=jax.ShapeDtypeStruct((_BH, _D), jnp.float32),
    )(pooled, W, b.reshape(1, _D))


def kernel(indices, table, W, b):
    idx_flat = indices.reshape(-1).astype(jnp.int32)
    outs = []
    for h in range(_NSPLIT):
        idx_h = idx_flat[h * _BH * _SEQ:(h + 1) * _BH * _SEQ]
        pooled = _gather_mean(idx_h, table)
        outs.append(_linear(pooled, W, b))
    return jnp.concatenate(outs, axis=0)


# speedup vs baseline: 1.0036x; 1.0036x over previous
import functools

import jax
import jax.numpy as jnp
from jax import lax
from jax.experimental import pallas as pl
from jax.experimental.pallas import tpu as pltpu
from jax.experimental.pallas import tpu_sc as plsc

_B = 4096
_SEQ = 50
_D = 128
_NC = 2
_NS = 16
_NW = _NC * _NS
_BPW = _B // _NW
_CHUNK = 8
_STEPS = _BPW // _CHUNK
_G = _CHUNK * _SEQ
_NLANE = _D // 16


def _gather_mean(idx_flat, table):
    mesh = plsc.VectorSubcoreMesh(core_axis_name="c", subcore_axis_name="s")

    @functools.partial(
        pl.kernel,
        mesh=mesh,
        out_type=jax.ShapeDtypeStruct((_B, _D), jnp.float32),
        scratch_types=[
            pltpu.VMEM((_G,), jnp.int32),
            pltpu.VMEM((_G,), jnp.int32),
            pltpu.VMEM((_G, _D), jnp.float32),
            pltpu.VMEM((_G, _D), jnp.float32),
            pltpu.VMEM((_CHUNK, _D), jnp.float32),
            pltpu.SemaphoreType.DMA,
            pltpu.SemaphoreType.DMA,
        ],
    )
    def k(idx_hbm, table_hbm, out_hbm, idx0, idx1, rows0, rows1, pooled_v,
          sem0, sem1):
        wid = lax.axis_index("s") * _NC + lax.axis_index("c")
        base = wid * _BPW

        def issue(s, idx_v, rows_v, sem):
            row0 = base + s * _CHUNK
            pltpu.sync_copy(idx_hbm.at[pl.ds(row0 * _SEQ, _G)], idx_v)
            pltpu.async_copy(table_hbm.at[idx_v], rows_v, sem)

        def wait(idx_v, rows_v, sem):
            pltpu.make_async_copy(table_hbm.at[idx_v], rows_v, sem).wait()

        def reduce_store(s, rows_v):
            for r in range(_CHUNK):
                def red(j, acc):
                    return tuple(
                        acc[c] + rows_v[r * _SEQ + j, pl.ds(c * 16, 16)]
                        for c in range(_NLANE)
                    )
                acc0 = tuple(jnp.zeros((16,), jnp.float32) for _ in range(_NLANE))
                acc = lax.fori_loop(0, _SEQ, red, acc0)
                for c in range(_NLANE):
                    pooled_v[r, pl.ds(c * 16, 16)] = acc[c] * (1.0 / _SEQ)
            pltpu.sync_copy(pooled_v, out_hbm.at[pl.ds(base + s * _CHUNK, _CHUNK)])

        issue(0, idx0, rows0, sem0)

        def pair(i, carry):
            s = 2 * i
            issue(s + 1, idx1, rows1, sem1)
            wait(idx0, rows0, sem0)
            reduce_store(s, rows0)

            @pl.when(s + 2 < _STEPS)
            def _():
                issue(s + 2, idx0, rows0, sem0)

            wait(idx1, rows1, sem1)
            reduce_store(s + 1, rows1)
            return carry

        lax.fori_loop(0, _STEPS // 2, pair, 0)

    return k(idx_flat, table)


def _linear(pooled, W, b):
    def mm(p_ref, w_ref, b_ref, o_ref):
        o_ref[...] = lax.dot_general(
            p_ref[...], w_ref[...], (((1,), (1,)), ((), ())),
            preferred_element_type=jnp.float32,
        ) + b_ref[...]

    return pl.pallas_call(
        mm,
        out_shape=jax.ShapeDtypeStruct((_B, _D), jnp.float32),
    )(pooled, W, b.reshape(1, _D))


def kernel(indices, table, W, b):
    idx_flat = indices.reshape(-1).astype(jnp.int32)
    pooled = _gather_mean(idx_flat, table)
    return _linear(pooled, W, b)


# speedup vs baseline: 1.0635x; 1.0596x over previous
import functools

import jax
import jax.numpy as jnp
from jax import lax
from jax.experimental import pallas as pl
from jax.experimental.pallas import tpu as pltpu
from jax.experimental.pallas import tpu_sc as plsc

_B = 4096
_SEQ = 50
_D = 128
_NC = 2
_NS = 16
_NW = _NC * _NS
_BPW = _B // _NW
_CHUNK = 8
_STEPS = _BPW // _CHUNK
_G = _CHUNK * _SEQ
_NLANE = _D // 16


def _gather_mean(idx_flat, table):
    mesh = plsc.VectorSubcoreMesh(core_axis_name="c", subcore_axis_name="s")

    @functools.partial(
        pl.kernel,
        mesh=mesh,
        out_type=jax.ShapeDtypeStruct((_B, _D), jnp.float32),
        scratch_types=[
            pltpu.VMEM((_BPW * _SEQ,), jnp.int32),
            pltpu.VMEM((_G, _D), jnp.float32),
            pltpu.VMEM((_G, _D), jnp.float32),
            pltpu.VMEM((_CHUNK, _D), jnp.float32),
            pltpu.SemaphoreType.DMA,
            pltpu.SemaphoreType.DMA,
        ],
    )
    def k(idx_hbm, table_hbm, out_hbm, idx_all, rows0, rows1, pooled_v,
          sem0, sem1):
        wid = lax.axis_index("s") * _NC + lax.axis_index("c")
        base = wid * _BPW

        pltpu.sync_copy(idx_hbm.at[pl.ds(base * _SEQ, _BPW * _SEQ)], idx_all)

        def issue(s, rows_v, sem):
            pltpu.async_copy(
                table_hbm.at[idx_all.at[pl.ds(s * _G, _G)]], rows_v, sem)

        def wait(s, rows_v, sem):
            pltpu.make_async_copy(
                table_hbm.at[idx_all.at[pl.ds(s * _G, _G)]], rows_v, sem
            ).wait()

        def reduce_store(s, rows_v):
            for r in range(_CHUNK):
                def red(j, acc):
                    return tuple(
                        acc[c] + rows_v[r * _SEQ + j, pl.ds(c * 16, 16)]
                        for c in range(_NLANE)
                    )
                acc0 = tuple(jnp.zeros((16,), jnp.float32) for _ in range(_NLANE))
                acc = lax.fori_loop(0, _SEQ, red, acc0)
                for c in range(_NLANE):
                    pooled_v[r, pl.ds(c * 16, 16)] = acc[c] * (1.0 / _SEQ)
            pltpu.sync_copy(pooled_v, out_hbm.at[pl.ds(base + s * _CHUNK, _CHUNK)])

        issue(0, rows0, sem0)

        def pair(i, carry):
            s = 2 * i
            issue(s + 1, rows1, sem1)
            wait(s, rows0, sem0)
            reduce_store(s, rows0)

            @pl.when(s + 2 < _STEPS)
            def _():
                issue(s + 2, rows0, sem0)

            wait(s + 1, rows1, sem1)
            reduce_store(s + 1, rows1)
            return carry

        lax.fori_loop(0, _STEPS // 2, pair, 0)

    return k(idx_flat, table)


def _linear(pooled, W, b):
    def mm(p_ref, w_ref, b_ref, o_ref):
        o_ref[...] = lax.dot_general(
            p_ref[...], w_ref[...], (((1,), (1,)), ((), ())),
            preferred_element_type=jnp.float32,
        ) + b_ref[...]

    return pl.pallas_call(
        mm,
        out_shape=jax.ShapeDtypeStruct((_B, _D), jnp.float32),
    )(pooled, W, b.reshape(1, _D))


def kernel(indices, table, W, b):
    idx_flat = indices.reshape(-1).astype(jnp.int32)
    pooled = _gather_mean(idx_flat, table)
    return _linear(pooled, W, b)


# speedup vs baseline: 1.0721x; 1.0081x over previous
import functools

import jax
import jax.numpy as jnp
from jax import lax
from jax.experimental import pallas as pl
from jax.experimental.pallas import tpu as pltpu
from jax.experimental.pallas import tpu_sc as plsc

_B = 4096
_SEQ = 50
_D = 128
_NC = 2
_NS = 16
_NW = _NC * _NS
_BPW = _B // _NW
_CHUNK = 8
_STEPS = _BPW // _CHUNK
_G = _CHUNK * _SEQ
_NLANE = _D // 16


def _gather_mean(idx_flat, table):
    mesh = plsc.VectorSubcoreMesh(core_axis_name="c", subcore_axis_name="s")

    @functools.partial(
        pl.kernel,
        mesh=mesh,
        out_type=jax.ShapeDtypeStruct((_B, _D), jnp.float32),
        scratch_types=[
            pltpu.VMEM((_BPW * _SEQ,), jnp.int32),
            pltpu.VMEM((_G, _D), jnp.float32),
            pltpu.VMEM((_G, _D), jnp.float32),
            pltpu.VMEM((_CHUNK, _D), jnp.float32),
            pltpu.VMEM((_CHUNK, _D), jnp.float32),
            pltpu.SemaphoreType.DMA,
            pltpu.SemaphoreType.DMA,
            pltpu.SemaphoreType.DMA,
            pltpu.SemaphoreType.DMA,
        ],
    )
    def k(idx_hbm, table_hbm, out_hbm, idx_all, rows0, rows1, pooled0,
          pooled1, sem0, sem1, semo0, semo1):
        wid = lax.axis_index("s") * _NC + lax.axis_index("c")
        base = wid * _BPW

        pltpu.sync_copy(idx_hbm.at[pl.ds(base * _SEQ, _BPW * _SEQ)], idx_all)

        def issue(s, rows_v, sem):
            pltpu.async_copy(
                table_hbm.at[idx_all.at[pl.ds(s * _G, _G)]], rows_v, sem)

        def wait(s, rows_v, sem):
            pltpu.make_async_copy(
                table_hbm.at[idx_all.at[pl.ds(s * _G, _G)]], rows_v, sem
            ).wait()

        def out_slot(s):
            return out_hbm.at[pl.ds(base + s * _CHUNK, _CHUNK)]

        def reduce_store(s, rows_v, pooled_v, semo):
            for r in range(_CHUNK):
                def red(j, acc):
                    return tuple(
                        acc[c] + rows_v[r * _SEQ + j, pl.ds(c * 16, 16)]
                        for c in range(_NLANE)
                    )
                acc0 = tuple(jnp.zeros((16,), jnp.float32) for _ in range(_NLANE))
                acc = lax.fori_loop(0, _SEQ, red, acc0)
                for c in range(_NLANE):
                    pooled_v[r, pl.ds(c * 16, 16)] = acc[c] * (1.0 / _SEQ)
            pltpu.async_copy(pooled_v, out_slot(s), semo)

        def wait_out(s, pooled_v, semo):
            pltpu.make_async_copy(pooled_v, out_slot(s), semo).wait()

        issue(0, rows0, sem0)

        def pair(i, carry):
            s = 2 * i
            issue(s + 1, rows1, sem1)
            wait(s, rows0, sem0)

            @pl.when(s >= 2)
            def _():
                wait_out(lax.max(s - 2, 0), pooled0, semo0)

            reduce_store(s, rows0, pooled0, semo0)

            @pl.when(s + 2 < _STEPS)
            def _():
                issue(s + 2, rows0, sem0)

            wait(s + 1, rows1, sem1)

            @pl.when(s >= 2)
            def _():
                wait_out(lax.max(s - 1, 0), pooled1, semo1)

            reduce_store(s + 1, rows1, pooled1, semo1)
            return carry

        lax.fori_loop(0, _STEPS // 2, pair, 0)
        wait_out(_STEPS - 2, pooled0, semo0)
        wait_out(_STEPS - 1, pooled1, semo1)

    return k(idx_flat, table)


def _linear(pooled, W, b):
    def mm(p_ref, w_ref, b_ref, o_ref):
        o_ref[...] = lax.dot_general(
            p_ref[...], w_ref[...], (((1,), (1,)), ((), ())),
            preferred_element_type=jnp.float32,
        ) + b_ref[...]

    return pl.pallas_call(
        mm,
        out_shape=jax.ShapeDtypeStruct((_B, _D), jnp.float32),
    )(pooled, W, b.reshape(1, _D))


def kernel(indices, table, W, b):
    idx_flat = indices.reshape(-1).astype(jnp.int32)
    pooled = _gather_mean(idx_flat, table)
    return _linear(pooled, W, b)
